# Initial kernel scaffold; baseline (speedup 1.0000x reference)
import jax
import jax.numpy as jnp
from jax.experimental import pallas as pl


def kernel(anchors, objectness, pred_bbox_deltas):
    raise NotImplementedError("write your pallas kernel here")


# speedup vs baseline: 3.8027x; 3.8027x over previous
import functools

import jax
import jax.numpy as jnp
import numpy as np
from jax import lax
from jax.experimental import pallas as pl
from jax.experimental.pallas import tpu as pltpu

_N = 20000
_ROWS = 160
_NPAD = _ROWS * 128
_PRE = 2000
_PRE_ROWS = 16
_PREPAD = _PRE_ROWS * 128
_POST = 1000
_OUT_ROWS = 8
_NMS_T = 0.7
_IMG_W = 1024.0
_IMG_H = 1024.0
_BBOX_CLIP = float(np.log(1000.0 / 16.0))
_NEG = float("-inf")


def _rpn_kernel(a0r, a1r, a2r, a3r, d0r, d1r, d2r, d3r, scr,
                o1r, o2r, o3r, o4r,
                x1s, y1s, x2s, y2s, ssr):
    a0 = a0r[...]
    a1 = a1r[...]
    a2 = a2r[...]
    a3 = a3r[...]
    widths = a2 - a0 + 1.0
    heights = a3 - a1 + 1.0
    ctr_x = a0 + 0.5 * widths
    ctr_y = a1 + 0.5 * heights
    dw = jnp.minimum(d2r[...], _BBOX_CLIP)
    dh = jnp.minimum(d3r[...], _BBOX_CLIP)
    pcx = d0r[...] * widths + ctr_x
    pcy = d1r[...] * heights + ctr_y
    pw = jnp.exp(dw) * widths
    ph = jnp.exp(dh) * heights
    x1s[...] = jnp.clip(pcx - 0.5 * pw, 0.0, _IMG_W - 1.0)
    y1s[...] = jnp.clip(pcy - 0.5 * ph, 0.0, _IMG_H - 1.0)
    x2s[...] = jnp.clip(pcx + 0.5 * pw - 1.0, 0.0, _IMG_W - 1.0)
    y2s[...] = jnp.clip(pcy + 0.5 * ph - 1.0, 0.0, _IMG_H - 1.0)
    ssr[...] = scr[...]

    flat_big = (lax.broadcasted_iota(jnp.int32, (_ROWS, 128), 0) * 128
                + lax.broadcasted_iota(jnp.int32, (_ROWS, 128), 1))
    flat_sm = (lax.broadcasted_iota(jnp.int32, (_PRE_ROWS, 128), 0) * 128
               + lax.broadcasted_iota(jnp.int32, (_PRE_ROWS, 128), 1))
    flat_out = (lax.broadcasted_iota(jnp.int32, (_OUT_ROWS, 128), 0) * 128
                + lax.broadcasted_iota(jnp.int32, (_OUT_ROWS, 128), 1))

    def p1(j, carry):
        bx1, by1, bx2, by2, bs = carry
        s = ssr[...]
        m = jnp.max(s)
        sel = jnp.min(jnp.where(s == m, flat_big, _NPAD))
        oh = flat_big == sel
        gx1 = jnp.sum(jnp.where(oh, x1s[...], 0.0))
        gy1 = jnp.sum(jnp.where(oh, y1s[...], 0.0))
        gx2 = jnp.sum(jnp.where(oh, x2s[...], 0.0))
        gy2 = jnp.sum(jnp.where(oh, y2s[...], 0.0))
        ssr[...] = jnp.where(oh, _NEG, s)
        tgt = flat_sm == j
        bx1 = jnp.where(tgt, gx1, bx1)
        by1 = jnp.where(tgt, gy1, by1)
        bx2 = jnp.where(tgt, gx2, bx2)
        by2 = jnp.where(tgt, gy2, by2)
        bs = jnp.where(tgt, m, bs)
        return (bx1, by1, bx2, by2, bs)

    zeros_sm = jnp.zeros((_PRE_ROWS, 128), jnp.float32)
    neg_sm = jnp.full((_PRE_ROWS, 128), _NEG, jnp.float32)
    bx1, by1, bx2, by2, bs = lax.fori_loop(
        0, _PRE, p1, (zeros_sm, zeros_sm, zeros_sm, zeros_sm, neg_sm))

    areas = (bx2 - bx1 + 1.0) * (by2 - by1 + 1.0)
    keep0 = jnp.where(flat_sm < _PRE, 1.0, 0.0)

    def p2(i, keep):
        oh = flat_sm == i
        xi1 = jnp.sum(jnp.where(oh, bx1, 0.0))
        yi1 = jnp.sum(jnp.where(oh, by1, 0.0))
        xi2 = jnp.sum(jnp.where(oh, bx2, 0.0))
        yi2 = jnp.sum(jnp.where(oh, by2, 0.0))
        ai = jnp.sum(jnp.where(oh, areas, 0.0))
        ki = jnp.sum(jnp.where(oh, keep, 0.0))
        xx1 = jnp.maximum(xi1, bx1)
        yy1 = jnp.maximum(yi1, by1)
        xx2 = jnp.minimum(xi2, bx2)
        yy2 = jnp.minimum(yi2, by2)
        w = jnp.maximum(xx2 - xx1 + 1.0, 0.0)
        h = jnp.maximum(yy2 - yy1 + 1.0, 0.0)
        inter = w * h
        iou = inter / (ai + areas - inter)
        supp = (iou > _NMS_T) & (flat_sm > i) & (ki > 0.5)
        return jnp.where(supp, 0.0, keep)

    keep = lax.fori_loop(0, _PRE, p2, keep0)

    masked0 = jnp.where(keep > 0.5, bs, _NEG)
    zeros_out = jnp.zeros((_OUT_ROWS, 128), jnp.float32)

    def p3(o, carry):
        m1, m2, m3, m4, msk = carry
        mm = jnp.max(msk)
        sel = jnp.min(jnp.where(msk == mm, flat_sm, _PREPAD))
        oh = flat_sm == sel
        g1 = jnp.sum(jnp.where(oh, bx1, 0.0))
        g2 = jnp.sum(jnp.where(oh, by1, 0.0))
        g3 = jnp.sum(jnp.where(oh, bx2, 0.0))
        g4 = jnp.sum(jnp.where(oh, by2, 0.0))
        msk = jnp.where(oh, _NEG, msk)
        tgt = flat_out == o
        m1 = jnp.where(tgt, g1, m1)
        m2 = jnp.where(tgt, g2, m2)
        m3 = jnp.where(tgt, g3, m3)
        m4 = jnp.where(tgt, g4, m4)
        return (m1, m2, m3, m4, msk)

    m1, m2, m3, m4, _ = lax.fori_loop(
        0, _POST, p3, (zeros_out, zeros_out, zeros_out, zeros_out, masked0))
    o1r[...] = m1
    o2r[...] = m2
    o3r[...] = m3
    o4r[...] = m4


@jax.jit
def kernel(anchors, objectness, pred_bbox_deltas):
    pad = _NPAD - _N
    s = jnp.pad(objectness.reshape(-1), (0, pad),
                constant_values=_NEG).reshape(_ROWS, 128)
    a = jnp.pad(anchors, ((0, pad), (0, 0)))
    d = jnp.pad(pred_bbox_deltas, ((0, pad), (0, 0)))
    cols = [a[:, i].reshape(_ROWS, 128) for i in range(4)]
    dcols = [d[:, i].reshape(_ROWS, 128) for i in range(4)]
    out_shape = [jax.ShapeDtypeStruct((_OUT_ROWS, 128), jnp.float32)] * 4
    scratch = [pltpu.VMEM((_ROWS, 128), jnp.float32)] * 5
    o1, o2, o3, o4 = pl.pallas_call(
        _rpn_kernel,
        out_shape=out_shape,
        scratch_shapes=scratch,
    )(*cols, *dcols, s)
    return jnp.stack([o.reshape(-1)[:_POST] for o in (o1, o2, o3, o4)],
                     axis=1)
